# baseline (device time: 17140 ns/iter reference)
import jax
import jax.numpy as jnp
from jax import lax
from jax.experimental import pallas as pl
from jax.experimental.pallas import tpu as pltpu

N_DEV = 4
N_PEERS = N_DEV - 1
N_LAYERS = 3
N_CHUNKS = 4


def kernel(x, Win0, Wout0, Win1, Wout1, Win2, Wout2):
    b, d = x.shape
    k = Win0.shape[1]
    cw = d // N_CHUNKS

    def body(x_hbm, win0_hbm, wout0_hbm, win1_hbm, wout1_hbm, win2_hbm,
             wout2_hbm, out_ref,
             x_v, win_v, wout_v, send_buf, recv_buf,
             load_sems, send_sems, recv_sems):
        my = lax.axis_index("i")

        win_hbm = [win0_hbm, win1_hbm, win2_hbm]
        wout_hbm = [wout0_hbm, wout1_hbm, wout2_hbm]
        x_load = pltpu.make_async_copy(x_hbm, x_v, load_sems.at[0])
        x_load.start()
        win_loads = []
        wout_loads = []
        hk = k // 2
        w0a = pltpu.make_async_copy(win0_hbm.at[:, pl.ds(0, hk)],
                                    win_v.at[0, :, pl.ds(0, hk)],
                                    load_sems.at[7])
        w0b = pltpu.make_async_copy(win0_hbm.at[:, pl.ds(hk, hk)],
                                    win_v.at[0, :, pl.ds(hk, hk)],
                                    load_sems.at[1])
        o0a = pltpu.make_async_copy(wout0_hbm.at[pl.ds(0, hk), :],
                                    wout_v.at[0, pl.ds(0, hk), :],
                                    load_sems.at[8])
        o0b = pltpu.make_async_copy(wout0_hbm.at[pl.ds(hk, hk), :],
                                    wout_v.at[0, pl.ds(hk, hk), :],
                                    load_sems.at[2])
        for dma in (w0a, w0b, o0a, o0b):
            dma.start()

        class _Pair:
            def __init__(self, a, b):
                self._a, self._b = a, b

            def wait(self):
                self._a.wait()
                self._b.wait()

        win_loads.append(_Pair(w0a, w0b))
        wout_loads.append(_Pair(o0a, o0b))
        for L in range(1, N_LAYERS):
            wl = pltpu.make_async_copy(win_hbm[L], win_v.at[L],
                                       load_sems.at[1 + 2 * L])
            ol = pltpu.make_async_copy(wout_hbm[L], wout_v.at[L],
                                       load_sems.at[2 + 2 * L])
            wl.start()
            ol.start()
            win_loads.append(wl)
            wout_loads.append(ol)

        barrier_sem = pltpu.get_barrier_semaphore()
        for idx in range(N_PEERS):
            j = (my + 1 + idx) % N_DEV
            pl.semaphore_signal(
                barrier_sem, inc=1,
                device_id=(j,), device_id_type=pl.DeviceIdType.MESH,
            )

        x_load.wait()
        win_loads[0].wait()
        h_pre = jnp.dot(x_v[:, :], win_v[0], preferred_element_type=jnp.float32)

        for L in range(N_LAYERS):
            h = jnp.maximum(h_pre, 0.0)
            wout_loads[L].wait()

            sends = []
            p_chunks = []
            for c in range(N_CHUNKS):
                cs = pl.ds(c * cw, cw)
                p_c = jnp.dot(h, wout_v[L, :, cs],
                              preferred_element_type=jnp.float32)
                p_chunks.append(p_c)
                send_buf[L, :, cs] = p_c.astype(jnp.bfloat16)
                if L == 0 and c == 0:
                    pl.semaphore_wait(barrier_sem, N_PEERS)
                for idx in range(N_PEERS):
                    j = (my + 1 + idx) % N_DEV
                    rdma = pltpu.make_async_remote_copy(
                        src_ref=send_buf.at[L, :, cs],
                        dst_ref=recv_buf.at[L, 2 - idx, :, cs],
                        send_sem=send_sems.at[idx, c],
                        recv_sem=recv_sems.at[L, 2 - idx, c],
                        device_id=(j,),
                        device_id_type=pl.DeviceIdType.MESH,
                    )
                    rdma.start()
                    sends.append(rdma)

            if L < N_LAYERS - 1:
                win_loads[L + 1].wait()

            h_pre_next = None
            for c in range(N_CHUNKS):
                cs = pl.ds(c * cw, cw)
                for r in range(N_PEERS):
                    recv = pltpu.make_async_remote_copy(
                        src_ref=send_buf.at[L, :, cs],
                        dst_ref=recv_buf.at[L, r, :, cs],
                        send_sem=send_sems.at[r, c],
                        recv_sem=recv_sems.at[L, r, c],
                        device_id=(my,),
                        device_id_type=pl.DeviceIdType.MESH,
                    )
                    recv.wait_recv()
                x_c = p_chunks[c]
                for r in range(N_PEERS):
                    x_c = x_c + recv_buf[L, r, :, cs].astype(jnp.float32)
                if L < N_LAYERS - 1:
                    g = jnp.dot(x_c, win_v[L + 1, cs, :],
                                preferred_element_type=jnp.float32)
                    h_pre_next = g if h_pre_next is None else h_pre_next + g
                else:
                    out_ref[:, cs] = x_c

            for rdma in sends:
                rdma.wait_send()
            h_pre = h_pre_next

    return pl.pallas_call(
        body,
        out_shape=jax.ShapeDtypeStruct((b, d), jnp.float32),
        in_specs=[pl.BlockSpec(memory_space=pltpu.MemorySpace.HBM)] * 7,
        out_specs=pl.BlockSpec(memory_space=pltpu.MemorySpace.VMEM),
        scratch_shapes=[
            pltpu.VMEM((b, d), jnp.float32),
            pltpu.VMEM((N_LAYERS, d, k), jnp.float32),
            pltpu.VMEM((N_LAYERS, k, d), jnp.float32),
            pltpu.VMEM((N_LAYERS, b, d), jnp.bfloat16),
            pltpu.VMEM((N_LAYERS, N_PEERS, b, d), jnp.bfloat16),
            pltpu.SemaphoreType.DMA((3 + 2 * N_LAYERS,)),
            pltpu.SemaphoreType.DMA((N_PEERS, N_CHUNKS)),
            pltpu.SemaphoreType.DMA((N_LAYERS, N_PEERS, N_CHUNKS)),
        ],
        compiler_params=pltpu.CompilerParams(collective_id=0),
    )(*[
        pltpu.with_memory_space_constraint(a, pltpu.MemorySpace.HBM)
        for a in (x, Win0, Wout0, Win1, Wout1, Win2, Wout2)
    ])


# device time: 17002 ns/iter; 1.0081x vs baseline; 1.0081x over previous
import jax
import jax.numpy as jnp
from jax import lax
from jax.experimental import pallas as pl
from jax.experimental.pallas import tpu as pltpu

N_DEV = 4
N_PEERS = N_DEV - 1
N_LAYERS = 3
N_CHUNKS = 1


def kernel(x, Win0, Wout0, Win1, Wout1, Win2, Wout2):
    b, d = x.shape
    k = Win0.shape[1]
    cw = d // N_CHUNKS

    def body(x_hbm, win0_hbm, wout0_hbm, win1_hbm, wout1_hbm, win2_hbm,
             wout2_hbm, out_ref,
             x_v, win_v, wout_v, send_buf, recv_buf,
             load_sems, send_sems, recv_sems):
        my = lax.axis_index("i")

        win_hbm = [win0_hbm, win1_hbm, win2_hbm]
        wout_hbm = [wout0_hbm, wout1_hbm, wout2_hbm]
        x_load = pltpu.make_async_copy(x_hbm, x_v, load_sems.at[0])
        x_load.start()
        win_loads = []
        wout_loads = []
        for L in range(N_LAYERS):
            wl = pltpu.make_async_copy(win_hbm[L], win_v.at[L],
                                       load_sems.at[1 + 2 * L])
            ol = pltpu.make_async_copy(wout_hbm[L], wout_v.at[L],
                                       load_sems.at[2 + 2 * L])
            wl.start()
            ol.start()
            win_loads.append(wl)
            wout_loads.append(ol)

        barrier_sem = pltpu.get_barrier_semaphore()
        for idx in range(N_PEERS):
            j = (my + 1 + idx) % N_DEV
            pl.semaphore_signal(
                barrier_sem, inc=1,
                device_id=(j,), device_id_type=pl.DeviceIdType.MESH,
            )

        x_load.wait()
        win_loads[0].wait()
        h_pre = jnp.dot(x_v[:, :], win_v[0], preferred_element_type=jnp.float32)

        for L in range(N_LAYERS):
            h = jnp.maximum(h_pre, 0.0)
            wout_loads[L].wait()

            sends = []
            p_chunks = []
            for c in range(N_CHUNKS):
                cs = pl.ds(c * cw, cw)
                p_c = jnp.dot(h, wout_v[L, :, cs],
                              preferred_element_type=jnp.float32)
                p_chunks.append(p_c)
                send_buf[L, :, cs] = p_c.astype(jnp.bfloat16)
                if L == 0 and c == 0:
                    pl.semaphore_wait(barrier_sem, N_PEERS)
                for idx in range(N_PEERS):
                    j = (my + 1 + idx) % N_DEV
                    rdma = pltpu.make_async_remote_copy(
                        src_ref=send_buf.at[L, :, cs],
                        dst_ref=recv_buf.at[L, 2 - idx, :, cs],
                        send_sem=send_sems.at[idx, c],
                        recv_sem=recv_sems.at[L, 2 - idx, c],
                        device_id=(j,),
                        device_id_type=pl.DeviceIdType.MESH,
                    )
                    rdma.start()
                    sends.append(rdma)

            if L < N_LAYERS - 1:
                win_loads[L + 1].wait()

            h_pre_next = None
            for c in range(N_CHUNKS):
                cs = pl.ds(c * cw, cw)
                for r in range(N_PEERS):
                    recv = pltpu.make_async_remote_copy(
                        src_ref=send_buf.at[L, :, cs],
                        dst_ref=recv_buf.at[L, r, :, cs],
                        send_sem=send_sems.at[r, c],
                        recv_sem=recv_sems.at[L, r, c],
                        device_id=(my,),
                        device_id_type=pl.DeviceIdType.MESH,
                    )
                    recv.wait_recv()
                x_c = p_chunks[c]
                for r in range(N_PEERS):
                    x_c = x_c + recv_buf[L, r, :, cs].astype(jnp.float32)
                if L < N_LAYERS - 1:
                    g = jnp.dot(x_c, win_v[L + 1, cs, :],
                                preferred_element_type=jnp.float32)
                    h_pre_next = g if h_pre_next is None else h_pre_next + g
                else:
                    out_ref[:, cs] = x_c

            for rdma in sends:
                rdma.wait_send()
            h_pre = h_pre_next

    return pl.pallas_call(
        body,
        out_shape=jax.ShapeDtypeStruct((b, d), jnp.float32),
        in_specs=[pl.BlockSpec(memory_space=pltpu.MemorySpace.HBM)] * 7,
        out_specs=pl.BlockSpec(memory_space=pltpu.MemorySpace.VMEM),
        scratch_shapes=[
            pltpu.VMEM((b, d), jnp.float32),
            pltpu.VMEM((N_LAYERS, d, k), jnp.float32),
            pltpu.VMEM((N_LAYERS, k, d), jnp.float32),
            pltpu.VMEM((N_LAYERS, b, d), jnp.bfloat16),
            pltpu.VMEM((N_LAYERS, N_PEERS, b, d), jnp.bfloat16),
            pltpu.SemaphoreType.DMA((1 + 2 * N_LAYERS,)),
            pltpu.SemaphoreType.DMA((N_PEERS, N_CHUNKS)),
            pltpu.SemaphoreType.DMA((N_LAYERS, N_PEERS, N_CHUNKS)),
        ],
        compiler_params=pltpu.CompilerParams(collective_id=0),
    )(*[
        pltpu.with_memory_space_constraint(a, pltpu.MemorySpace.HBM)
        for a in (x, Win0, Wout0, Win1, Wout1, Win2, Wout2)
    ])
